# baseline (speedup 1.0000x reference)
import functools
import math

import jax
import jax.numpy as jnp
from jax import lax
from jax.experimental import pallas as pl
from jax.experimental.pallas import tpu as pltpu
from jax.experimental.pallas import tpu_sc as plsc

N = 10000
E = 160000
D = 256
DE = 16
H = 8
HD = D // H

NT = 10
TN = N // NT
W = 200
CH = 40
NCH = W // CH
EPT = E // 16
NWIN = EPT // W
HPC = H // 2


def _k1_body(x_ref, w_ref, b_ref, qkv_ref, qe_ref):
    t = jnp.dot(x_ref[...], w_ref[...], preferred_element_type=jnp.float32)
    qkv_ref[...] = t
    qe_ref[...] = jnp.dot(t[:, :D], b_ref[...], preferred_element_type=jnp.float32)


def _k3a_body(accv_ref, acce_ref, denr_ref, x_ref, webd_ref, woutt_ref, bo_ref,
              o1_ref, sums_ref):
    num = accv_ref[...] + jnp.dot(acce_ref[...], webd_ref[...],
                                  preferred_element_type=jnp.float32)
    agg = num / (denr_ref[...] + 1e-16)
    o1 = (jnp.dot(agg, woutt_ref[...], preferred_element_type=jnp.float32)
          + bo_ref[...] + x_ref[...])
    o1_ref[...] = o1

    @pl.when(pl.program_id(0) == 0)
    def _():
        sums_ref[...] = jnp.zeros_like(sums_ref)

    sums_ref[0:1, :] += jnp.sum(o1, axis=0, keepdims=True)
    sums_ref[1:2, :] += jnp.sum(o1 * o1, axis=0, keepdims=True)


def _k3b_body(o1_ref, sums_ref, gnw_ref, gnb_ref, gms_ref, out_ref):
    o1 = o1_ref[...]
    ms = gms_ref[...]
    mean = sums_ref[0:1, :] * (1.0 / N)
    var = sums_ref[1:2, :] * (1.0 / N) - (2.0 - ms) * ms * mean * mean
    cen = o1 - ms * mean
    y = gnw_ref[...] * cen / jnp.sqrt(var + 1e-5) + gnb_ref[...]
    out_ref[...] = y * 0.5 * (1.0 + lax.erf(y * (1.0 / math.sqrt(2.0))))


def _sc_body(qs_hbm, kv_hbm, ea_hbm, src_hbm, dst_hbm, zeros_hbm, acc_hbm,
             sacc, vqs0, vqs1, vkv0, vkv1, vea0, vea1, vsrc0, vsrc1,
             vdst0, vdst1, vdsts0, vdsts1, vmsg0, vmsg1,
             semi0, semi1, semg0, semg1, semsc0, semsc1, semv0, semv1):
    c = lax.axis_index("c")
    s = lax.axis_index("s")
    lane0 = lax.iota(jnp.int32, 16) == 0
    vqs = (vqs0, vqs1)
    vkv = (vkv0, vkv1)
    vea = (vea0, vea1)
    vsrc = (vsrc0, vsrc1)
    vdst = (vdst0, vdst1)
    vdsts = (vdsts0, vdsts1)
    vmsg = (vmsg0, vmsg1)
    semi = (semi0, semi1)
    semg = (semg0, semg1)
    semsc = (semsc0, semsc1)
    semv = (semv0, semv1)

    def idx_copies(w, b):
        row0 = s * (EPT // CH) + w * NCH
        return (pltpu.make_async_copy(src_hbm.at[pl.ds(row0, NCH), :],
                                      vsrc[b], semi[b]),
                pltpu.make_async_copy(dst_hbm.at[pl.ds(row0, NCH), :],
                                      vdst[b], semi[b]))

    def vdsts_copy(w, b):
        row0 = s * (EPT // CH) + w * NCH
        return pltpu.make_async_copy(dst_hbm.at[pl.ds(row0, NCH), :],
                                     vdsts[b], semv[b])

    def ea_copy(w, b):
        base = s * EPT + w * W
        return pltpu.make_async_copy(ea_hbm.at[pl.ds(base, W), :],
                                     vea[b], semi[b])

    def gather_copies(head, b):
        cps = []
        for j in range(NCH):
            cps.append(pltpu.make_async_copy(
                qs_hbm.at[head].at[vdst[b].at[j]],
                vqs[b].at[pl.ds(j * CH, CH), :], semg[b]))
            cps.append(pltpu.make_async_copy(
                kv_hbm.at[head].at[vsrc[b].at[j]],
                vkv[b].at[pl.ds(j * CH, CH), :], semg[b]))
        return cps

    def scatter_copies(b):
        return [pltpu.make_async_copy(vmsg[b].at[pl.ds(j * CH, CH), :],
                                      sacc.at[vdsts[b].at[j]], semsc[b])
                for j in range(NCH)]

    def compute(b):
        @plsc.parallel_loop(0, W, unroll=4)
        def _(w):
            q0 = vqs[b][w, 0:16]
            q1 = vqs[b][w, 16:32]
            qe = vqs[b][w, 32:48]
            k0 = vkv[b][w, 0:16]
            k1 = vkv[b][w, 16:32]
            v0 = vkv[b][w, 32:48]
            v1 = vkv[b][w, 48:64]
            ea = vea[b][w, :]
            p = q0 * k0 + q1 * k1 + qe * ea
            tot = plsc.cumsum(p)[15]
            ex = jnp.exp(jnp.full((16,), tot, jnp.float32))
            vmsg[b][w, 0:16] = ex * v0
            vmsg[b][w, 16:32] = ex * v1
            vmsg[b][w, 32:48] = ex * ea
            vmsg[b][w, 48:64] = jnp.where(lane0, ex, 0.0)

    for hh in range(HPC):
        head = c * HPC + hh

        @pl.when(s < NT)
        def _():
            r0 = s * TN
            pltpu.sync_copy(zeros_hbm.at[pl.ds(r0, TN), :],
                            sacc.at[pl.ds(r0, TN), :])

        plsc.subcore_barrier()

        for cp in idx_copies(0, 0):
            cp.start()
        ea_copy(0, 0).start()
        for cp in idx_copies(0, 0):
            cp.wait()
        ea_copy(0, 0).wait()
        for cp in gather_copies(head, 0):
            cp.start()
        for cp in idx_copies(1, 1):
            cp.start()
        ea_copy(1, 1).start()

        def pair(i, _):
            def one(w, b):
                o = b ^ 1
                for cp in gather_copies(head, b):
                    cp.wait()

                @pl.when(w >= 2)
                def _():
                    for cp in scatter_copies(b):
                        cp.wait()

                vdsts_copy(w, b).start()

                @pl.when(w + 2 < NWIN)
                def _():
                    for cp in idx_copies(w + 2, b):
                        cp.start()

                compute(b)

                @pl.when(w + 2 < NWIN)
                def _():
                    ea_copy(w + 2, b).start()

                vdsts_copy(w, b).wait()
                for cp in scatter_copies(b):
                    cp.start(add=True)

                @pl.when(w + 1 < NWIN)
                def _():
                    for cp in idx_copies(w + 1, o):
                        cp.wait()
                    ea_copy(w + 1, o).wait()
                    for cp in gather_copies(head, o):
                        cp.start()

            one(2 * i, 0)
            one(2 * i + 1, 1)
            return 0

        lax.fori_loop(0, NWIN // 2, pair, 0)
        for cp in scatter_copies(0):
            cp.wait()
        for cp in scatter_copies(1):
            cp.wait()
        plsc.subcore_barrier()

        @pl.when(s < NT)
        def _():
            r0 = s * TN
            pltpu.sync_copy(sacc.at[pl.ds(r0, TN), :],
                            acc_hbm.at[pl.ds(r0, TN), head, :])

        plsc.subcore_barrier()


def _sc_call(qs, kv, ea, src, dst, zeros):
    mesh = plsc.VectorSubcoreMesh(core_axis_name="c", subcore_axis_name="s")
    return pl.kernel(
        _sc_body,
        out_type=jax.ShapeDtypeStruct((N, H, 64), jnp.float32),
        mesh=mesh,
        compiler_params=pltpu.CompilerParams(
            needs_layout_passes=False, use_tc_tiling_on_sc=False),
        scratch_types=[
            pltpu.VMEM_SHARED((N, 64), jnp.float32),
            pltpu.VMEM((W, 48), jnp.float32),
            pltpu.VMEM((W, 48), jnp.float32),
            pltpu.VMEM((W, 64), jnp.float32),
            pltpu.VMEM((W, 64), jnp.float32),
            pltpu.VMEM((W, DE), jnp.float32),
            pltpu.VMEM((W, DE), jnp.float32),
            pltpu.VMEM((NCH, CH), jnp.int32),
            pltpu.VMEM((NCH, CH), jnp.int32),
            pltpu.VMEM((NCH, CH), jnp.int32),
            pltpu.VMEM((NCH, CH), jnp.int32),
            pltpu.VMEM((NCH, CH), jnp.int32),
            pltpu.VMEM((NCH, CH), jnp.int32),
            pltpu.VMEM((W, 64), jnp.float32),
            pltpu.VMEM((W, 64), jnp.float32),
            pltpu.SemaphoreType.DMA,
            pltpu.SemaphoreType.DMA,
            pltpu.SemaphoreType.DMA,
            pltpu.SemaphoreType.DMA,
            pltpu.SemaphoreType.DMA,
            pltpu.SemaphoreType.DMA,
            pltpu.SemaphoreType.DMA,
            pltpu.SemaphoreType.DMA,
        ],
    )(qs, kv, ea, src, dst, zeros)


def kernel(x, edge_index, edge_attr, WQ, WK, WV, WE, W_out, b_out,
           gn_weight, gn_bias, gn_mean_scale):
    scale = 1.0 / math.sqrt(HD)
    we_h = WE.reshape(H, HD, DE)
    eye = jnp.eye(H, dtype=jnp.float32)
    B = (eye[:, None, :, None] * we_h[:, :, None, :]).reshape(D, H * DE)
    WEbd = (eye[:, None, :, None] * jnp.transpose(we_h, (0, 2, 1))[:, :, None, :]
            ).reshape(H * DE, D)

    wall = jnp.concatenate([WQ.T * scale, WK.T, WV.T], axis=1)

    qkv, qe = pl.pallas_call(
        _k1_body,
        grid=(NT,),
        in_specs=[
            pl.BlockSpec((TN, D), lambda i: (i, 0)),
            pl.BlockSpec((D, 3 * D), lambda i: (0, 0)),
            pl.BlockSpec((D, H * DE), lambda i: (0, 0)),
        ],
        out_specs=[
            pl.BlockSpec((TN, 3 * D), lambda i: (i, 0)),
            pl.BlockSpec((TN, H * DE), lambda i: (i, 0)),
        ],
        out_shape=[
            jax.ShapeDtypeStruct((N, 3 * D), jnp.float32),
            jax.ShapeDtypeStruct((N, H * DE), jnp.float32),
        ],
    )(x, wall, B)

    q = qkv[:, :D].reshape(N, H, HD)
    k = qkv[:, D:2 * D].reshape(N, H, HD)
    v = qkv[:, 2 * D:].reshape(N, H, HD)
    qs = jnp.transpose(
        jnp.concatenate([q, qe.reshape(N, H, DE)], axis=2), (1, 0, 2))
    kv = jnp.transpose(jnp.concatenate([k, v], axis=2), (1, 0, 2))

    src = edge_index[0].reshape(E // CH, CH)
    dst = edge_index[1].reshape(E // CH, CH)
    zeros = jnp.zeros((N, 64), jnp.float32)

    acc = _sc_call(qs, kv, edge_attr, src, dst, zeros)

    accv = acc[:, :, :HD].reshape(N, D)
    acce = acc[:, :, HD:HD + DE].reshape(N, H * DE)
    denr = jnp.repeat(acc[:, :, HD + DE], HD, axis=1)

    o1, sums = pl.pallas_call(
        _k3a_body,
        grid=(NT,),
        in_specs=[
            pl.BlockSpec((TN, D), lambda i: (i, 0)),
            pl.BlockSpec((TN, H * DE), lambda i: (i, 0)),
            pl.BlockSpec((TN, D), lambda i: (i, 0)),
            pl.BlockSpec((TN, D), lambda i: (i, 0)),
            pl.BlockSpec((H * DE, D), lambda i: (0, 0)),
            pl.BlockSpec((D, D), lambda i: (0, 0)),
            pl.BlockSpec((1, D), lambda i: (0, 0)),
        ],
        out_specs=[
            pl.BlockSpec((TN, D), lambda i: (i, 0)),
            pl.BlockSpec((2, D), lambda i: (0, 0)),
        ],
        out_shape=[
            jax.ShapeDtypeStruct((N, D), jnp.float32),
            jax.ShapeDtypeStruct((2, D), jnp.float32),
        ],
    )(accv, acce, denr, x, WEbd, W_out.T, b_out.reshape(1, D))

    out = pl.pallas_call(
        _k3b_body,
        grid=(NT,),
        in_specs=[
            pl.BlockSpec((TN, D), lambda i: (i, 0)),
            pl.BlockSpec((2, D), lambda i: (0, 0)),
            pl.BlockSpec((1, D), lambda i: (0, 0)),
            pl.BlockSpec((1, D), lambda i: (0, 0)),
            pl.BlockSpec((1, D), lambda i: (0, 0)),
        ],
        out_specs=pl.BlockSpec((TN, D), lambda i: (i, 0)),
        out_shape=jax.ShapeDtypeStruct((N, D), jnp.float32),
    )(o1, sums, gn_weight.reshape(1, D), gn_bias.reshape(1, D),
      gn_mean_scale.reshape(1, D))
    return out


# speedup vs baseline: 1.1853x; 1.1853x over previous
import functools
import math

import jax
import jax.numpy as jnp
from jax import lax
from jax.experimental import pallas as pl
from jax.experimental.pallas import tpu as pltpu
from jax.experimental.pallas import tpu_sc as plsc

N = 10000
E = 160000
D = 256
DE = 16
H = 8
HD = D // H

NT = 10
TN = N // NT
W = 200
CH = 40
NCH = W // CH
EPT = E // 16
NWIN = EPT // W
HPC = H // 2


def _k1_body(x_ref, w_ref, b_ref, qkv_ref, qe_ref):
    t = jnp.dot(x_ref[...], w_ref[...], preferred_element_type=jnp.float32)
    qkv_ref[...] = t
    qe_ref[...] = jnp.dot(t[:, :D], b_ref[...], preferred_element_type=jnp.float32)


def _k3a_body(accv_ref, acce_ref, denr_ref, x_ref, webd_ref, woutt_ref, bo_ref,
              o1_ref, sums_ref):
    num = accv_ref[...] + jnp.dot(acce_ref[...], webd_ref[...],
                                  preferred_element_type=jnp.float32)
    agg = num / (denr_ref[...] + 1e-16)
    o1 = (jnp.dot(agg, woutt_ref[...], preferred_element_type=jnp.float32)
          + bo_ref[...] + x_ref[...])
    o1_ref[...] = o1

    @pl.when(pl.program_id(0) == 0)
    def _():
        sums_ref[...] = jnp.zeros_like(sums_ref)

    sums_ref[0:1, :] += jnp.sum(o1, axis=0, keepdims=True)
    sums_ref[1:2, :] += jnp.sum(o1 * o1, axis=0, keepdims=True)


def _k3b_body(o1_ref, sums_ref, gnw_ref, gnb_ref, gms_ref, out_ref):
    o1 = o1_ref[...]
    ms = gms_ref[...]
    mean = sums_ref[0:1, :] * (1.0 / N)
    var = sums_ref[1:2, :] * (1.0 / N) - (2.0 - ms) * ms * mean * mean
    cen = o1 - ms * mean
    y = gnw_ref[...] * cen / jnp.sqrt(var + 1e-5) + gnb_ref[...]
    out_ref[...] = y * 0.5 * (1.0 + lax.erf(y * (1.0 / math.sqrt(2.0))))


def _sc_body(qs_hbm, kv_hbm, ea_hbm, src_hbm, dst_hbm, zeros_hbm, acc_hbm,
             sacc, vqs0, vqs1, vkv0, vkv1, vea0, vea1, vsrc0, vsrc1,
             vdst0, vdst1, vdsts0, vdsts1, vmsg0, vmsg1,
             semi0, semi1, semg0, semg1, semsc0, semsc1, semv0, semv1):
    c = lax.axis_index("c")
    s = lax.axis_index("s")
    lane0 = lax.iota(jnp.int32, 16) == 0
    vqs = (vqs0, vqs1)
    vkv = (vkv0, vkv1)
    vea = (vea0, vea1)
    vsrc = (vsrc0, vsrc1)
    vdst = (vdst0, vdst1)
    vdsts = (vdsts0, vdsts1)
    vmsg = (vmsg0, vmsg1)
    semi = (semi0, semi1)
    semg = (semg0, semg1)
    semsc = (semsc0, semsc1)
    semv = (semv0, semv1)

    def idx_copies(w, b):
        row0 = s * (EPT // CH) + w * NCH
        return (pltpu.make_async_copy(src_hbm.at[pl.ds(row0, NCH), :],
                                      vsrc[b], semi[b]),
                pltpu.make_async_copy(dst_hbm.at[pl.ds(row0, NCH), :],
                                      vdst[b], semi[b]))

    def vdsts_copy(w, b):
        row0 = s * (EPT // CH) + w * NCH
        return pltpu.make_async_copy(dst_hbm.at[pl.ds(row0, NCH), :],
                                     vdsts[b], semv[b])

    def ea_copy(w, b):
        base = s * EPT + w * W
        return pltpu.make_async_copy(ea_hbm.at[pl.ds(base, W), :],
                                     vea[b], semi[b])

    def gather_copies(head, b):
        cps = []
        for j in range(NCH):
            cps.append(pltpu.make_async_copy(
                qs_hbm.at[head].at[vdst[b].at[j]],
                vqs[b].at[pl.ds(j * CH, CH), :], semg[b]))
            cps.append(pltpu.make_async_copy(
                kv_hbm.at[head].at[vsrc[b].at[j]],
                vkv[b].at[pl.ds(j * CH, CH), :], semg[b]))
        return cps

    def scatter_copies(b):
        return [pltpu.make_async_copy(vmsg[b].at[pl.ds(j * CH, CH), :],
                                      sacc.at[vdsts[b].at[j]], semsc[b])
                for j in range(NCH)]

    def compute(b):
        @plsc.parallel_loop(0, W, unroll=4)
        def _(w):
            q0 = vqs[b][w, 0:16]
            q1 = vqs[b][w, 16:32]
            qe = vqs[b][w, 32:48]
            k0 = vkv[b][w, 0:16]
            k1 = vkv[b][w, 16:32]
            v0 = vkv[b][w, 32:48]
            v1 = vkv[b][w, 48:64]
            ea = vea[b][w, :]
            p = q0 * k0 + q1 * k1 + qe * ea
            tot = plsc.cumsum(p)[15]
            ex = jnp.exp(jnp.full((16,), tot, jnp.float32))
            vmsg[b][w, 0:16] = ex * v0
            vmsg[b][w, 16:32] = ex * v1
            vmsg[b][w, 32:48] = ex * ea
            vmsg[b][w, 48:64] = jnp.where(lane0, ex, 0.0)

    for hh in range(HPC):
        head = c * HPC + hh

        @pl.when(s < NT)
        def _():
            r0 = s * TN
            pltpu.sync_copy(zeros_hbm.at[pl.ds(r0, TN), :],
                            sacc.at[pl.ds(r0, TN), :])

        plsc.subcore_barrier()

        for cp in idx_copies(0, 0):
            cp.start()
        ea_copy(0, 0).start()
        for cp in idx_copies(0, 0):
            cp.wait()
        ea_copy(0, 0).wait()
        for cp in gather_copies(head, 0):
            cp.start()
        for cp in idx_copies(1, 1):
            cp.start()
        ea_copy(1, 1).start()

        def pair(i, _):
            def one(w, b):
                o = b ^ 1
                for cp in gather_copies(head, b):
                    cp.wait()

                @pl.when(w >= 2)
                def _():
                    for cp in scatter_copies(b):
                        cp.wait()

                vdsts_copy(w, b).start()

                @pl.when(w + 2 < NWIN)
                def _():
                    for cp in idx_copies(w + 2, b):
                        cp.start()

                @pl.when(w + 1 < NWIN)
                def _():
                    for cp in idx_copies(w + 1, o):
                        cp.wait()
                    ea_copy(w + 1, o).wait()
                    for cp in gather_copies(head, o):
                        cp.start()

                compute(b)

                @pl.when(w + 2 < NWIN)
                def _():
                    ea_copy(w + 2, b).start()

                vdsts_copy(w, b).wait()
                for cp in scatter_copies(b):
                    cp.start(add=True)

            one(2 * i, 0)
            one(2 * i + 1, 1)
            return 0

        lax.fori_loop(0, NWIN // 2, pair, 0)
        for cp in scatter_copies(0):
            cp.wait()
        for cp in scatter_copies(1):
            cp.wait()
        plsc.subcore_barrier()

        @pl.when(s < NT)
        def _():
            r0 = s * TN
            pltpu.sync_copy(sacc.at[pl.ds(r0, TN), :],
                            acc_hbm.at[pl.ds(r0, TN), head, :])

        plsc.subcore_barrier()


def _sc_call(qs, kv, ea, src, dst, zeros):
    mesh = plsc.VectorSubcoreMesh(core_axis_name="c", subcore_axis_name="s")
    return pl.kernel(
        _sc_body,
        out_type=jax.ShapeDtypeStruct((N, H, 64), jnp.float32),
        mesh=mesh,
        compiler_params=pltpu.CompilerParams(
            needs_layout_passes=False, use_tc_tiling_on_sc=False),
        scratch_types=[
            pltpu.VMEM_SHARED((N, 64), jnp.float32),
            pltpu.VMEM((W, 48), jnp.float32),
            pltpu.VMEM((W, 48), jnp.float32),
            pltpu.VMEM((W, 64), jnp.float32),
            pltpu.VMEM((W, 64), jnp.float32),
            pltpu.VMEM((W, DE), jnp.float32),
            pltpu.VMEM((W, DE), jnp.float32),
            pltpu.VMEM((NCH, CH), jnp.int32),
            pltpu.VMEM((NCH, CH), jnp.int32),
            pltpu.VMEM((NCH, CH), jnp.int32),
            pltpu.VMEM((NCH, CH), jnp.int32),
            pltpu.VMEM((NCH, CH), jnp.int32),
            pltpu.VMEM((NCH, CH), jnp.int32),
            pltpu.VMEM((W, 64), jnp.float32),
            pltpu.VMEM((W, 64), jnp.float32),
            pltpu.SemaphoreType.DMA,
            pltpu.SemaphoreType.DMA,
            pltpu.SemaphoreType.DMA,
            pltpu.SemaphoreType.DMA,
            pltpu.SemaphoreType.DMA,
            pltpu.SemaphoreType.DMA,
            pltpu.SemaphoreType.DMA,
            pltpu.SemaphoreType.DMA,
        ],
    )(qs, kv, ea, src, dst, zeros)


def kernel(x, edge_index, edge_attr, WQ, WK, WV, WE, W_out, b_out,
           gn_weight, gn_bias, gn_mean_scale):
    scale = 1.0 / math.sqrt(HD)
    we_h = WE.reshape(H, HD, DE)
    eye = jnp.eye(H, dtype=jnp.float32)
    B = (eye[:, None, :, None] * we_h[:, :, None, :]).reshape(D, H * DE)
    WEbd = (eye[:, None, :, None] * jnp.transpose(we_h, (0, 2, 1))[:, :, None, :]
            ).reshape(H * DE, D)

    wall = jnp.concatenate([WQ.T * scale, WK.T, WV.T], axis=1)

    qkv, qe = pl.pallas_call(
        _k1_body,
        grid=(NT,),
        in_specs=[
            pl.BlockSpec((TN, D), lambda i: (i, 0)),
            pl.BlockSpec((D, 3 * D), lambda i: (0, 0)),
            pl.BlockSpec((D, H * DE), lambda i: (0, 0)),
        ],
        out_specs=[
            pl.BlockSpec((TN, 3 * D), lambda i: (i, 0)),
            pl.BlockSpec((TN, H * DE), lambda i: (i, 0)),
        ],
        out_shape=[
            jax.ShapeDtypeStruct((N, 3 * D), jnp.float32),
            jax.ShapeDtypeStruct((N, H * DE), jnp.float32),
        ],
    )(x, wall, B)

    q = qkv[:, :D].reshape(N, H, HD)
    k = qkv[:, D:2 * D].reshape(N, H, HD)
    v = qkv[:, 2 * D:].reshape(N, H, HD)
    qs = jnp.transpose(
        jnp.concatenate([q, qe.reshape(N, H, DE)], axis=2), (1, 0, 2))
    kv = jnp.transpose(jnp.concatenate([k, v], axis=2), (1, 0, 2))

    src = edge_index[0].reshape(E // CH, CH)
    dst = edge_index[1].reshape(E // CH, CH)
    zeros = jnp.zeros((N, 64), jnp.float32)

    acc = _sc_call(qs, kv, edge_attr, src, dst, zeros)

    accv = acc[:, :, :HD].reshape(N, D)
    acce = acc[:, :, HD:HD + DE].reshape(N, H * DE)
    denr = jnp.repeat(acc[:, :, HD + DE], HD, axis=1)

    o1, sums = pl.pallas_call(
        _k3a_body,
        grid=(NT,),
        in_specs=[
            pl.BlockSpec((TN, D), lambda i: (i, 0)),
            pl.BlockSpec((TN, H * DE), lambda i: (i, 0)),
            pl.BlockSpec((TN, D), lambda i: (i, 0)),
            pl.BlockSpec((TN, D), lambda i: (i, 0)),
            pl.BlockSpec((H * DE, D), lambda i: (0, 0)),
            pl.BlockSpec((D, D), lambda i: (0, 0)),
            pl.BlockSpec((1, D), lambda i: (0, 0)),
        ],
        out_specs=[
            pl.BlockSpec((TN, D), lambda i: (i, 0)),
            pl.BlockSpec((2, D), lambda i: (0, 0)),
        ],
        out_shape=[
            jax.ShapeDtypeStruct((N, D), jnp.float32),
            jax.ShapeDtypeStruct((2, D), jnp.float32),
        ],
    )(accv, acce, denr, x, WEbd, W_out.T, b_out.reshape(1, D))

    out = pl.pallas_call(
        _k3b_body,
        grid=(NT,),
        in_specs=[
            pl.BlockSpec((TN, D), lambda i: (i, 0)),
            pl.BlockSpec((2, D), lambda i: (0, 0)),
            pl.BlockSpec((1, D), lambda i: (0, 0)),
            pl.BlockSpec((1, D), lambda i: (0, 0)),
            pl.BlockSpec((1, D), lambda i: (0, 0)),
        ],
        out_specs=pl.BlockSpec((TN, D), lambda i: (i, 0)),
        out_shape=jax.ShapeDtypeStruct((N, D), jnp.float32),
    )(o1, sums, gn_weight.reshape(1, D), gn_bias.reshape(1, D),
      gn_mean_scale.reshape(1, D))
    return out


# speedup vs baseline: 1.3819x; 1.1659x over previous
import functools
import math

import jax
import jax.numpy as jnp
from jax import lax
from jax.experimental import pallas as pl
from jax.experimental.pallas import tpu as pltpu
from jax.experimental.pallas import tpu_sc as plsc

N = 10000
E = 160000
D = 256
DE = 16
H = 8
HD = D // H

NT = 10
TN = N // NT
W = 200
CH = 40
NCH = W // CH
EPT = E // 16
NWIN = EPT // W
HPC = H // 2


def _k1_body(x_ref, w_ref, b_ref, qkv_ref, qe_ref):
    t = jnp.dot(x_ref[...], w_ref[...], preferred_element_type=jnp.float32)
    qkv_ref[...] = t
    qe_ref[...] = jnp.dot(t[:, :D], b_ref[...], preferred_element_type=jnp.float32)


def _k3a_body(accv_ref, acce_ref, den_ref, x_ref, webd_ref, woutt_ref, bo_ref,
              r_ref, o1_ref, sums_ref):
    num = accv_ref[...] + jnp.dot(acce_ref[...], webd_ref[...],
                                  preferred_element_type=jnp.float32)
    denr = jnp.dot(den_ref[...], r_ref[...],
                   preferred_element_type=jnp.float32)
    agg = num / (denr + 1e-16)
    o1 = (jnp.dot(agg, woutt_ref[...], preferred_element_type=jnp.float32)
          + bo_ref[...] + x_ref[...])
    o1_ref[...] = o1

    @pl.when(pl.program_id(0) == 0)
    def _():
        sums_ref[...] = jnp.zeros_like(sums_ref)

    sums_ref[0:1, :] += jnp.sum(o1, axis=0, keepdims=True)
    sums_ref[1:2, :] += jnp.sum(o1 * o1, axis=0, keepdims=True)


def _k3b_body(o1_ref, sums_ref, gnw_ref, gnb_ref, gms_ref, out_ref):
    o1 = o1_ref[...]
    ms = gms_ref[...]
    mean = sums_ref[0:1, :] * (1.0 / N)
    var = sums_ref[1:2, :] * (1.0 / N) - (2.0 - ms) * ms * mean * mean
    cen = o1 - ms * mean
    y = gnw_ref[...] * cen / jnp.sqrt(var + 1e-5) + gnb_ref[...]
    out_ref[...] = y * 0.5 * (1.0 + lax.erf(y * (1.0 / math.sqrt(2.0))))


def _sc_body(qs_hbm, kv_hbm, ea_hbm, src_hbm, dst_hbm,
             accv_hbm, acce_hbm, den_hbm,
             sacc, vqs0, vqs1, vkv0, vkv1, vea0, vea1, vsrc0, vsrc1,
             vdst0, vdst1, vdsts0, vdsts1, vmsg0, vmsg1,
             semi0, semi1, semg0, semg1, semsc0, semsc1, semv0, semv1):
    c = lax.axis_index("c")
    s = lax.axis_index("s")
    lane0 = lax.iota(jnp.int32, 16) == 0
    vqs = (vqs0, vqs1)
    vkv = (vkv0, vkv1)
    vea = (vea0, vea1)
    vsrc = (vsrc0, vsrc1)
    vdst = (vdst0, vdst1)
    vdsts = (vdsts0, vdsts1)
    vmsg = (vmsg0, vmsg1)
    semi = (semi0, semi1)
    semg = (semg0, semg1)
    semsc = (semsc0, semsc1)
    semv = (semv0, semv1)

    def idx_copies(w, b):
        row0 = s * (EPT // CH) + w * NCH
        return (pltpu.make_async_copy(src_hbm.at[pl.ds(row0, NCH), :],
                                      vsrc[b], semi[b]),
                pltpu.make_async_copy(dst_hbm.at[pl.ds(row0, NCH), :],
                                      vdst[b], semi[b]))

    def vdsts_copy(w, b):
        row0 = s * (EPT // CH) + w * NCH
        return pltpu.make_async_copy(dst_hbm.at[pl.ds(row0, NCH), :],
                                     vdsts[b], semv[b])

    def ea_copy(w, b):
        base = s * EPT + w * W
        return pltpu.make_async_copy(ea_hbm.at[pl.ds(base, W), :],
                                     vea[b], semi[b])

    def gather_copies(head, b):
        cps = []
        for j in range(NCH):
            cps.append(pltpu.make_async_copy(
                qs_hbm.at[head].at[vdst[b].at[j]],
                vqs[b].at[pl.ds(j * CH, CH), :], semg[b]))
            cps.append(pltpu.make_async_copy(
                kv_hbm.at[head].at[vsrc[b].at[j]],
                vkv[b].at[pl.ds(j * CH, CH), :], semg[b]))
        return cps

    def scatter_copies(b):
        return [pltpu.make_async_copy(vmsg[b].at[pl.ds(j * CH, CH), :],
                                      sacc.at[vdsts[b].at[j]], semsc[b])
                for j in range(NCH)]

    def compute(b):
        @plsc.parallel_loop(0, W, unroll=4)
        def _(w):
            q0 = vqs[b][w, 0:16]
            q1 = vqs[b][w, 16:32]
            qe = vqs[b][w, 32:48]
            k0 = vkv[b][w, 0:16]
            k1 = vkv[b][w, 16:32]
            v0 = vkv[b][w, 32:48]
            v1 = vkv[b][w, 48:64]
            ea = vea[b][w, :]
            p = q0 * k0 + q1 * k1 + qe * ea
            tot = plsc.cumsum(p)[15]
            ex = jnp.exp(jnp.full((16,), tot, jnp.float32))
            vmsg[b][w, 0:16] = ex * v0
            vmsg[b][w, 16:32] = ex * v1
            vmsg[b][w, 32:48] = ex * ea
            vmsg[b][w, 48:64] = jnp.where(lane0, ex, 0.0)

    for hh in range(HPC):
        head = c * HPC + hh

        @plsc.parallel_loop(0, W)
        def _(w):
            z = jnp.zeros((16,), jnp.float32)
            vmsg0[w, 0:16] = z
            vmsg0[w, 16:32] = z
            vmsg0[w, 32:48] = z
            vmsg0[w, 48:64] = z

        @pl.when(s < NT)
        def _():
            r0 = s * TN
            for j in range(TN // W):
                pltpu.sync_copy(vmsg0, sacc.at[pl.ds(r0 + j * W, W), :])

        plsc.subcore_barrier()

        for cp in idx_copies(0, 0):
            cp.start()
        ea_copy(0, 0).start()
        for cp in idx_copies(0, 0):
            cp.wait()
        ea_copy(0, 0).wait()
        for cp in gather_copies(head, 0):
            cp.start()
        for cp in idx_copies(1, 1):
            cp.start()
        ea_copy(1, 1).start()

        def pair(i, _):
            def one(w, b):
                o = b ^ 1
                for cp in gather_copies(head, b):
                    cp.wait()

                @pl.when(w >= 2)
                def _():
                    for cp in scatter_copies(b):
                        cp.wait()

                vdsts_copy(w, b).start()

                @pl.when(w + 2 < NWIN)
                def _():
                    for cp in idx_copies(w + 2, b):
                        cp.start()

                @pl.when(w + 1 < NWIN)
                def _():
                    for cp in idx_copies(w + 1, o):
                        cp.wait()
                    ea_copy(w + 1, o).wait()
                    for cp in gather_copies(head, o):
                        cp.start()

                compute(b)

                @pl.when(w + 2 < NWIN)
                def _():
                    ea_copy(w + 2, b).start()

                vdsts_copy(w, b).wait()
                for cp in scatter_copies(b):
                    cp.start(add=True)

            one(2 * i, 0)
            one(2 * i + 1, 1)
            return 0

        lax.fori_loop(0, NWIN // 2, pair, 0)
        for cp in scatter_copies(0):
            cp.wait()
        for cp in scatter_copies(1):
            cp.wait()
        plsc.subcore_barrier()

        @pl.when(s < NT)
        def _():
            r0 = s * TN
            pltpu.sync_copy(sacc.at[pl.ds(r0, TN), 0:32],
                            accv_hbm.at[pl.ds(r0, TN), pl.ds(head * HD, HD)])
            pltpu.sync_copy(sacc.at[pl.ds(r0, TN), 32:48],
                            acce_hbm.at[pl.ds(r0, TN), pl.ds(head * DE, DE)])
            pltpu.sync_copy(sacc.at[pl.ds(r0, TN), 48:56],
                            den_hbm.at[pl.ds(r0, TN), pl.ds(head * 8, 8)])

        plsc.subcore_barrier()


def _sc_call(qs, kv, ea, src, dst):
    mesh = plsc.VectorSubcoreMesh(core_axis_name="c", subcore_axis_name="s")
    return pl.kernel(
        _sc_body,
        out_type=(jax.ShapeDtypeStruct((N, D), jnp.float32),
                  jax.ShapeDtypeStruct((N, H * DE), jnp.float32),
                  jax.ShapeDtypeStruct((N, H * 8), jnp.float32)),
        mesh=mesh,
        compiler_params=pltpu.CompilerParams(
            needs_layout_passes=False, use_tc_tiling_on_sc=False),
        scratch_types=[
            pltpu.VMEM_SHARED((N, 64), jnp.float32),
            pltpu.VMEM((W, 48), jnp.float32),
            pltpu.VMEM((W, 48), jnp.float32),
            pltpu.VMEM((W, 64), jnp.float32),
            pltpu.VMEM((W, 64), jnp.float32),
            pltpu.VMEM((W, DE), jnp.float32),
            pltpu.VMEM((W, DE), jnp.float32),
            pltpu.VMEM((NCH, CH), jnp.int32),
            pltpu.VMEM((NCH, CH), jnp.int32),
            pltpu.VMEM((NCH, CH), jnp.int32),
            pltpu.VMEM((NCH, CH), jnp.int32),
            pltpu.VMEM((NCH, CH), jnp.int32),
            pltpu.VMEM((NCH, CH), jnp.int32),
            pltpu.VMEM((W, 64), jnp.float32),
            pltpu.VMEM((W, 64), jnp.float32),
            pltpu.SemaphoreType.DMA,
            pltpu.SemaphoreType.DMA,
            pltpu.SemaphoreType.DMA,
            pltpu.SemaphoreType.DMA,
            pltpu.SemaphoreType.DMA,
            pltpu.SemaphoreType.DMA,
            pltpu.SemaphoreType.DMA,
            pltpu.SemaphoreType.DMA,
        ],
    )(qs, kv, ea, src, dst)


def kernel(x, edge_index, edge_attr, WQ, WK, WV, WE, W_out, b_out,
           gn_weight, gn_bias, gn_mean_scale):
    scale = 1.0 / math.sqrt(HD)
    we_h = WE.reshape(H, HD, DE)
    eye = jnp.eye(H, dtype=jnp.float32)
    B = (eye[:, None, :, None] * we_h[:, :, None, :]).reshape(D, H * DE)
    WEbd = (eye[:, None, :, None] * jnp.transpose(we_h, (0, 2, 1))[:, :, None, :]
            ).reshape(H * DE, D)

    wall = jnp.concatenate([WQ.T * scale, WK.T, WV.T], axis=1)

    qkv, qe = pl.pallas_call(
        _k1_body,
        grid=(NT,),
        in_specs=[
            pl.BlockSpec((TN, D), lambda i: (i, 0)),
            pl.BlockSpec((D, 3 * D), lambda i: (0, 0)),
            pl.BlockSpec((D, H * DE), lambda i: (0, 0)),
        ],
        out_specs=[
            pl.BlockSpec((TN, 3 * D), lambda i: (i, 0)),
            pl.BlockSpec((TN, H * DE), lambda i: (i, 0)),
        ],
        out_shape=[
            jax.ShapeDtypeStruct((N, 3 * D), jnp.float32),
            jax.ShapeDtypeStruct((N, H * DE), jnp.float32),
        ],
    )(x, wall, B)

    q = qkv[:, :D].reshape(N, H, HD)
    k = qkv[:, D:2 * D].reshape(N, H, HD)
    v = qkv[:, 2 * D:].reshape(N, H, HD)
    qs = jnp.transpose(
        jnp.concatenate([q, qe.reshape(N, H, DE)], axis=2), (1, 0, 2))
    kv = jnp.transpose(jnp.concatenate([k, v], axis=2), (1, 0, 2))

    src = edge_index[0].reshape(E // CH, CH)
    dst = edge_index[1].reshape(E // CH, CH)

    accv, acce, den = _sc_call(qs, kv, edge_attr, src, dst)
    sel = jnp.zeros((8, HD), jnp.float32).at[0, :].set(1.0)
    rrep = jnp.kron(jnp.eye(H, dtype=jnp.float32), sel)

    o1, sums = pl.pallas_call(
        _k3a_body,
        grid=(NT,),
        in_specs=[
            pl.BlockSpec((TN, D), lambda i: (i, 0)),
            pl.BlockSpec((TN, H * DE), lambda i: (i, 0)),
            pl.BlockSpec((TN, H * 8), lambda i: (i, 0)),
            pl.BlockSpec((TN, D), lambda i: (i, 0)),
            pl.BlockSpec((H * DE, D), lambda i: (0, 0)),
            pl.BlockSpec((D, D), lambda i: (0, 0)),
            pl.BlockSpec((1, D), lambda i: (0, 0)),
            pl.BlockSpec((H * 8, D), lambda i: (0, 0)),
        ],
        out_specs=[
            pl.BlockSpec((TN, D), lambda i: (i, 0)),
            pl.BlockSpec((2, D), lambda i: (0, 0)),
        ],
        out_shape=[
            jax.ShapeDtypeStruct((N, D), jnp.float32),
            jax.ShapeDtypeStruct((2, D), jnp.float32),
        ],
    )(accv, acce, den, x, WEbd, W_out.T, b_out.reshape(1, D), rrep)

    out = pl.pallas_call(
        _k3b_body,
        grid=(NT,),
        in_specs=[
            pl.BlockSpec((TN, D), lambda i: (i, 0)),
            pl.BlockSpec((2, D), lambda i: (0, 0)),
            pl.BlockSpec((1, D), lambda i: (0, 0)),
            pl.BlockSpec((1, D), lambda i: (0, 0)),
            pl.BlockSpec((1, D), lambda i: (0, 0)),
        ],
        out_specs=pl.BlockSpec((TN, D), lambda i: (i, 0)),
        out_shape=jax.ShapeDtypeStruct((N, D), jnp.float32),
    )(o1, sums, gn_weight.reshape(1, D), gn_bias.reshape(1, D),
      gn_mean_scale.reshape(1, D))
    return out
